# baseline (speedup 1.0000x reference)
import functools

import jax
import jax.numpy as jnp
from jax import lax
from jax.experimental import pallas as pl
from jax.experimental.pallas import tpu as pltpu
from jax.experimental.pallas import tpu_sc as plsc

N_NODES = 10000
N_EDGES = 320000
N_FEAT = 128
N_META_IN = 16
N_OUT = 9

NC = 2
NS = 16
LANES = 16
NP = 10240
SLICE = NP // NS
EC = N_EDGES // NS
EG = EC // LANES
RED_G = SLICE // LANES


def _proj_body(x1_ref, x2_ref, w1_ref, w2_ref, w3_ref, w4_ref, v1_ref, v2_ref):
    wc = lax.dot_general(w3_ref[...], w4_ref[...], (((1,), (0,)), ((), ())),
                         precision=lax.Precision.HIGHEST)
    wc = lax.dot_general(w2_ref[...], wc, (((1,), (0,)), ((), ())),
                         precision=lax.Precision.HIGHEST)
    wc = lax.dot_general(w1_ref[...], wc, (((1,), (0,)), ((), ())),
                         precision=lax.Precision.HIGHEST)
    v1_ref[...] = lax.dot_general(x1_ref[...], wc, (((1,), (0,)), ((), ())),
                                  precision=lax.Precision.HIGHEST)
    v2_ref[...] = lax.dot_general(x2_ref[...], wc, (((1,), (0,)), ((), ())),
                                  precision=lax.Precision.HIGHEST)


def _project(x1, x2, W1, W2, W3, W4):
    return pl.pallas_call(
        _proj_body,
        out_shape=(jax.ShapeDtypeStruct((N_NODES, 1), jnp.float32),
                   jax.ShapeDtypeStruct((N_NODES, 1), jnp.float32)),
    )(x1, x2, W1, W2, W3, W4)


def _fast_rsqrt(x):
    i = plsc.bitcast(x, jnp.int32)
    i = jnp.int32(0x5F3759DF) - lax.shift_right_logical(i, 1)
    y = plsc.bitcast(i, jnp.float32)
    for _ in range(3):
        y = y * (1.5 - 0.5 * x * y * y)
    return y


def _gcn_sc_body(v_hbm, src_hbm, dst_hbm, cvec_hbm, z_hbm,
                 src_v, dst_v, norm_v, v_loc, dinv_loc, acc, tmp, sbuf, cbuf,
                 sh_mat, sh_vec):
    cid = lax.axis_index("c")
    sid = lax.axis_index("s")
    ebase = cid * N_EDGES + sid * EC
    zeros16 = jnp.zeros((LANES,), jnp.float32)
    ones16 = jnp.ones((LANES,), jnp.float32)

    pltpu.sync_copy(src_hbm.at[pl.ds(ebase, EC)], src_v)
    pltpu.sync_copy(dst_hbm.at[pl.ds(ebase, EC)], dst_v)
    pltpu.sync_copy(v_hbm.at[pl.ds(cid * N_NODES, N_NODES)],
                    v_loc.at[pl.ds(0, N_NODES)])
    pltpu.sync_copy(cvec_hbm, cbuf)
    for j in range(N_NODES // LANES, NP // LANES):
        v_loc[pl.ds(j * LANES, LANES)] = zeros16

    def _zero_acc():
        def body(i, _):
            acc[pl.ds(i * LANES, LANES)] = zeros16
            return 0
        lax.fori_loop(0, NP // LANES, body, 0)

    def _reduce_tiles():
        pltpu.sync_copy(acc, sh_mat.at[sid])
        plsc.subcore_barrier()
        pltpu.sync_copy(sh_mat.at[:, pl.ds(sid * SLICE, SLICE)], tmp)

    def _publish_and_fetch(dst_loc):
        pltpu.sync_copy(sbuf, sh_vec.at[pl.ds(sid * SLICE, SLICE)])
        plsc.subcore_barrier()
        pltpu.sync_copy(sh_vec, dst_loc)

    _zero_acc()

    def deg_body(i, _):
        d16 = dst_v[pl.ds(i * LANES, LANES)]
        plsc.addupdate_scatter(acc, [d16], ones16)
        return 0
    lax.fori_loop(0, EG, deg_body, 0)
    _reduce_tiles()

    def deg_red(g, _):
        off = g * LANES
        s = tmp[0, pl.ds(off, LANES)]
        for r in range(1, NS):
            s = s + tmp[r, pl.ds(off, LANES)]
        sbuf[pl.ds(off, LANES)] = _fast_rsqrt(s + 1.0)
        return 0
    lax.fori_loop(0, RED_G, deg_red, 0)
    _publish_and_fetch(dinv_loc)

    def norm_body(i, _):
        off = i * LANES
        s16 = src_v[pl.ds(off, LANES)]
        d16 = dst_v[pl.ds(off, LANES)]
        norm_v[pl.ds(off, LANES)] = (plsc.load_gather(dinv_loc, [s16]) *
                                     plsc.load_gather(dinv_loc, [d16]))
        return 0
    lax.fori_loop(0, EG, norm_body, 0)

    for layer in range(4):
        _zero_acc()

        def edge_body(i, _):
            off = i * LANES
            s16 = src_v[pl.ds(off, LANES)]
            vv = plsc.load_gather(v_loc, [s16])
            nv = norm_v[pl.ds(off, LANES)]
            d16 = dst_v[pl.ds(off, LANES)]
            plsc.addupdate_scatter(acc, [d16], vv * nv)
            return 0
        lax.fori_loop(0, EG, edge_body, 0)
        _reduce_tiles()
        cv = cbuf[pl.ds(layer * LANES, LANES)]

        def app_red(g, _):
            off = g * LANES
            s = tmp[0, pl.ds(off, LANES)]
            for r in range(1, NS):
                s = s + tmp[r, pl.ds(off, LANES)]
            goff = sid * SLICE + off
            dv = dinv_loc[pl.ds(goff, LANES)]
            vv = v_loc[pl.ds(goff, LANES)]
            sbuf[pl.ds(off, LANES)] = s + dv * dv * vv + cv
            return 0
        lax.fori_loop(0, RED_G, app_red, 0)
        if layer < 3:
            _publish_and_fetch(v_loc)
        else:
            pltpu.sync_copy(sbuf, sh_vec.at[pl.ds(sid * SLICE, SLICE)])
            plsc.subcore_barrier()

    @pl.when(sid == 0)
    def _():
        pltpu.sync_copy(sh_vec.at[pl.ds(0, N_NODES)],
                        acc.at[pl.ds(0, N_NODES)])
        pltpu.sync_copy(acc.at[pl.ds(0, N_NODES)],
                        z_hbm.at[pl.ds(cid * N_NODES, N_NODES)])


@functools.lru_cache(maxsize=1)
def _gcn_sc():
  return pl.kernel(
    _gcn_sc_body,
    out_type=jax.ShapeDtypeStruct((NC * N_NODES,), jnp.float32),
    mesh=plsc.VectorSubcoreMesh(core_axis_name="c", subcore_axis_name="s",
                                num_cores=NC, num_subcores=NS),
    compiler_params=pltpu.CompilerParams(needs_layout_passes=False),
    scratch_types=[
        pltpu.VMEM((EC,), jnp.int32),
        pltpu.VMEM((EC,), jnp.int32),
        pltpu.VMEM((EC,), jnp.float32),
        pltpu.VMEM((NP,), jnp.float32),
        pltpu.VMEM((NP,), jnp.float32),
        pltpu.VMEM((NP,), jnp.float32),
        pltpu.VMEM((NS, SLICE), jnp.float32),
        pltpu.VMEM((SLICE,), jnp.float32),
        pltpu.VMEM((4 * LANES,), jnp.float32),
        pltpu.VMEM_SHARED((NS, NP), jnp.float32),
        pltpu.VMEM_SHARED((NP,), jnp.float32),
    ],
  )


RB = 400
NRB = (2 * N_NODES) // RB


def _bmat_body(wi_hbm, wfm_ref, b_ref, buf0, buf1, sem0, sem1):
    i = pl.program_id(0)

    def cp(blk, buf, sem):
        return pltpu.make_async_copy(wi_hbm.at[pl.ds(blk * RB, RB), :],
                                     buf, sem)

    @pl.when(i == 0)
    def _():
        cp(0, buf0, sem0).start()

    @pl.when((i + 1 < NRB) & (lax.rem(i, 2) == 0))
    def _():
        cp(i + 1, buf1, sem1).start()

    @pl.when((i + 1 < NRB) & (lax.rem(i, 2) == 1))
    def _():
        cp(i + 1, buf0, sem0).start()

    @pl.when(lax.rem(i, 2) == 0)
    def _():
        cp(i, buf0, sem0).wait()
        b_ref[...] = lax.dot_general(buf0[...], wfm_ref[...],
                                     (((1,), (0,)), ((), ())),
                                     precision=lax.Precision.DEFAULT)

    @pl.when(lax.rem(i, 2) == 1)
    def _():
        cp(i, buf1, sem1).wait()
        b_ref[...] = lax.dot_general(buf1[...], wfm_ref[...],
                                     (((1,), (0,)), ((), ())),
                                     precision=lax.Precision.DEFAULT)


def _bmat(Wi, Wf_main):
    return pl.pallas_call(
        _bmat_body,
        grid=(NRB,),
        in_specs=[
            pl.BlockSpec(memory_space=pl.ANY),
            pl.BlockSpec((N_NODES, N_OUT), lambda i: (0, 0)),
        ],
        out_specs=pl.BlockSpec((RB, N_OUT), lambda i: (i, 0)),
        out_shape=jax.ShapeDtypeStruct((2 * N_NODES, N_OUT), jnp.float32),
        scratch_shapes=[pltpu.VMEM((RB, N_NODES), jnp.float32),
                        pltpu.VMEM((RB, N_NODES), jnp.float32),
                        pltpu.SemaphoreType.DMA,
                        pltpu.SemaphoreType.DMA],
        compiler_params=pltpu.CompilerParams(
            dimension_semantics=("arbitrary",)),
    )(Wi, Wf_main)


def _tail_body(z_ref, b_ref, bi_ref, wfm_ref, meta_ref, wfmeta_ref, bf_ref,
               out_ref):
    r9 = lax.dot_general(z_ref[...], b_ref[...], (((1,), (0,)), ((), ())),
                         precision=lax.Precision.HIGHEST)
    r9 = r9 + lax.dot_general(bi_ref[...], wfm_ref[...],
                              (((1,), (0,)), ((), ())),
                              precision=lax.Precision.HIGHEST)
    r9 = r9 + lax.dot_general(meta_ref[...], wfmeta_ref[...],
                              (((1,), (0,)), ((), ())),
                              precision=lax.Precision.HIGHEST)
    out_ref[...] = r9 + bf_ref[...]


def _tail(z_row, B, bi, Wf_main, meta, Wf_meta, bf):
    return pl.pallas_call(
        _tail_body,
        out_shape=jax.ShapeDtypeStruct((1, N_OUT), jnp.float32),
    )(z_row, B, bi, Wf_main, meta, Wf_meta, bf)


def kernel(x1, edge_index1, x2, edge_index2, meta, W1, b1, W2, b2, W3, b3,
           W4, b4, Wi, bi, Wf, bf):
    c1 = (b1 @ W2 @ W3 @ W4)[0]
    c2 = (b2 @ W3 @ W4)[0]
    c3 = (b3 @ W4)[0]
    c4 = b4[0]
    cvec = jnp.concatenate([jnp.full((LANES,), c, jnp.float32)
                            for c in (c1, c2, c3, c4)])

    v1, v2 = _project(x1, x2, W1, W2, W3, W4)
    vflat = jnp.concatenate([v1[:, 0], v2[:, 0]])
    src2 = jnp.concatenate([edge_index1[0], edge_index2[0]])
    dst2 = jnp.concatenate([edge_index1[1], edge_index2[1]])

    z2d = _gcn_sc()(vflat, src2, dst2, cvec)

    B = _bmat(Wi, Wf[:N_NODES])
    out = _tail(z2d.reshape(1, 2 * N_NODES), B, bi.reshape(1, N_NODES),
                Wf[:N_NODES], meta.reshape(1, N_META_IN), Wf[N_NODES:],
                bf.reshape(1, N_OUT))
    return out[0]


# speedup vs baseline: 2.3279x; 2.3279x over previous
import functools

import jax
import jax.numpy as jnp
from jax import lax
from jax.experimental import pallas as pl
from jax.experimental.pallas import tpu as pltpu
from jax.experimental.pallas import tpu_sc as plsc

N_NODES = 10000
N_EDGES = 320000
N_FEAT = 128
N_META_IN = 16
N_OUT = 9

NC = 2
NS = 16
LANES = 16
NP = 10240
SLICE = NP // NS
EC = N_EDGES // NS
EG = EC // LANES
RED_G = SLICE // LANES


def _proj_body(x1_ref, x2_ref, w1_ref, w2_ref, w3_ref, w4_ref, v1_ref, v2_ref):
    wc = lax.dot_general(w3_ref[...], w4_ref[...], (((1,), (0,)), ((), ())),
                         precision=lax.Precision.HIGHEST)
    wc = lax.dot_general(w2_ref[...], wc, (((1,), (0,)), ((), ())),
                         precision=lax.Precision.HIGHEST)
    wc = lax.dot_general(w1_ref[...], wc, (((1,), (0,)), ((), ())),
                         precision=lax.Precision.HIGHEST)
    v1_ref[...] = lax.dot_general(x1_ref[...], wc, (((1,), (0,)), ((), ())),
                                  precision=lax.Precision.HIGHEST)
    v2_ref[...] = lax.dot_general(x2_ref[...], wc, (((1,), (0,)), ((), ())),
                                  precision=lax.Precision.HIGHEST)


def _project(x1, x2, W1, W2, W3, W4):
    return pl.pallas_call(
        _proj_body,
        out_shape=(jax.ShapeDtypeStruct((N_NODES, 1), jnp.float32),
                   jax.ShapeDtypeStruct((N_NODES, 1), jnp.float32)),
    )(x1, x2, W1, W2, W3, W4)


def _fast_rsqrt(x):
    i = plsc.bitcast(x, jnp.int32)
    i = jnp.int32(0x5F3759DF) - lax.shift_right_logical(i, 1)
    y = plsc.bitcast(i, jnp.float32)
    for _ in range(3):
        y = y * (1.5 - 0.5 * x * y * y)
    return y


def _gcn_sc_body(v_hbm, src_hbm, dst_hbm, cvec_hbm, z_hbm,
                 src_v, dst_v, norm_v, v_loc, dinv_loc, acc, tmp, sbuf, cbuf,
                 sh_mat, sh_vec):
    cid = lax.axis_index("c")
    sid = lax.axis_index("s")
    ebase = cid * N_EDGES + sid * EC
    zeros16 = jnp.zeros((LANES,), jnp.float32)
    ones16 = jnp.ones((LANES,), jnp.float32)

    pltpu.sync_copy(src_hbm.at[pl.ds(ebase, EC)], src_v)
    pltpu.sync_copy(dst_hbm.at[pl.ds(ebase, EC)], dst_v)
    pltpu.sync_copy(v_hbm.at[pl.ds(cid * N_NODES, N_NODES)],
                    v_loc.at[pl.ds(0, N_NODES)])
    pltpu.sync_copy(cvec_hbm, cbuf)
    for j in range(N_NODES // LANES, NP // LANES):
        v_loc[pl.ds(j * LANES, LANES)] = zeros16

    def _zero_acc():
        def body(i, _):
            acc[pl.ds(i * LANES, LANES)] = zeros16
            return 0
        lax.fori_loop(0, NP // LANES, body, 0)

    def _reduce_tiles():
        pltpu.sync_copy(acc, sh_mat.at[sid])
        plsc.subcore_barrier()
        pltpu.sync_copy(sh_mat.at[:, pl.ds(sid * SLICE, SLICE)], tmp)

    def _publish_and_fetch(dst_loc):
        pltpu.sync_copy(sbuf, sh_vec.at[pl.ds(sid * SLICE, SLICE)])
        plsc.subcore_barrier()
        pltpu.sync_copy(sh_vec, dst_loc)

    _zero_acc()

    def deg_body(i, _):
        d16 = dst_v[pl.ds(i * LANES, LANES)]
        plsc.addupdate_scatter(acc, [d16], ones16)
        return 0
    lax.fori_loop(0, EG, deg_body, 0)
    _reduce_tiles()

    def deg_red(g, _):
        off = g * LANES
        s = tmp[0, pl.ds(off, LANES)]
        for r in range(1, NS):
            s = s + tmp[r, pl.ds(off, LANES)]
        sbuf[pl.ds(off, LANES)] = _fast_rsqrt(s + 1.0)
        return 0
    lax.fori_loop(0, RED_G, deg_red, 0)
    _publish_and_fetch(dinv_loc)

    def norm_body(i, _):
        off = i * LANES
        s16 = src_v[pl.ds(off, LANES)]
        d16 = dst_v[pl.ds(off, LANES)]
        norm_v[pl.ds(off, LANES)] = (plsc.load_gather(dinv_loc, [s16]) *
                                     plsc.load_gather(dinv_loc, [d16]))
        return 0
    lax.fori_loop(0, EG, norm_body, 0)

    for layer in range(4):
        _zero_acc()

        def edge_body(i, _):
            off = i * LANES
            s16 = src_v[pl.ds(off, LANES)]
            vv = plsc.load_gather(v_loc, [s16])
            nv = norm_v[pl.ds(off, LANES)]
            d16 = dst_v[pl.ds(off, LANES)]
            plsc.addupdate_scatter(acc, [d16], vv * nv)
            return 0
        lax.fori_loop(0, EG, edge_body, 0)
        _reduce_tiles()
        cv = cbuf[pl.ds(layer * LANES, LANES)]

        def app_red(g, _):
            off = g * LANES
            s = tmp[0, pl.ds(off, LANES)]
            for r in range(1, NS):
                s = s + tmp[r, pl.ds(off, LANES)]
            goff = sid * SLICE + off
            dv = dinv_loc[pl.ds(goff, LANES)]
            vv = v_loc[pl.ds(goff, LANES)]
            sbuf[pl.ds(off, LANES)] = s + dv * dv * vv + cv
            return 0
        lax.fori_loop(0, RED_G, app_red, 0)
        if layer < 3:
            _publish_and_fetch(v_loc)
        else:
            pltpu.sync_copy(sbuf, sh_vec.at[pl.ds(sid * SLICE, SLICE)])
            plsc.subcore_barrier()

    @pl.when(sid == 0)
    def _():
        pltpu.sync_copy(sh_vec.at[pl.ds(0, N_NODES)],
                        acc.at[pl.ds(0, N_NODES)])
        pltpu.sync_copy(acc.at[pl.ds(0, N_NODES)],
                        z_hbm.at[pl.ds(cid * N_NODES, N_NODES)])


@functools.lru_cache(maxsize=1)
def _gcn_sc():
  return pl.kernel(
    _gcn_sc_body,
    out_type=jax.ShapeDtypeStruct((NC * N_NODES,), jnp.float32),
    mesh=plsc.VectorSubcoreMesh(core_axis_name="c", subcore_axis_name="s",
                                num_cores=NC, num_subcores=NS),
    compiler_params=pltpu.CompilerParams(needs_layout_passes=False),
    scratch_types=[
        pltpu.VMEM((EC,), jnp.int32),
        pltpu.VMEM((EC,), jnp.int32),
        pltpu.VMEM((EC,), jnp.float32),
        pltpu.VMEM((NP,), jnp.float32),
        pltpu.VMEM((NP,), jnp.float32),
        pltpu.VMEM((NP,), jnp.float32),
        pltpu.VMEM((NS, SLICE), jnp.float32),
        pltpu.VMEM((SLICE,), jnp.float32),
        pltpu.VMEM((4 * LANES,), jnp.float32),
        pltpu.VMEM_SHARED((NS, NP), jnp.float32),
        pltpu.VMEM_SHARED((NP,), jnp.float32),
    ],
  )


RB = 400
NRB = (2 * N_NODES) // RB


RJ = 200


def _ri_body(wit_ref, z_ref, ri_ref):
    ri_ref[...] = lax.dot_general(wit_ref[...], z_ref[...],
                                  (((1,), (0,)), ((), ())),
                                  precision=lax.Precision.DEFAULT)


def _ri(WiT, z_col):
    return pl.pallas_call(
        _ri_body,
        grid=(N_NODES // RJ,),
        in_specs=[
            pl.BlockSpec((RJ, 2 * N_NODES), lambda i: (i, 0)),
            pl.BlockSpec((2 * N_NODES, 1), lambda i: (0, 0)),
        ],
        out_specs=pl.BlockSpec((RJ, 1), lambda i: (i, 0)),
        out_shape=jax.ShapeDtypeStruct((N_NODES, 1), jnp.float32),
        compiler_params=pltpu.CompilerParams(
            dimension_semantics=("arbitrary",)),
    )(WiT, z_col)


def _tail_body(ri_ref, bi_ref, wfm_ref, meta_ref, wfmeta_ref, bf_ref,
               out_ref):
    row = ri_ref[...] + bi_ref[...]
    r9 = lax.dot_general(row, wfm_ref[...], (((1,), (0,)), ((), ())),
                         precision=lax.Precision.HIGHEST)
    r9 = r9 + lax.dot_general(meta_ref[...], wfmeta_ref[...],
                              (((1,), (0,)), ((), ())),
                              precision=lax.Precision.HIGHEST)
    out_ref[...] = r9 + bf_ref[...]


def _tail(ri_row, bi, Wf_main, meta, Wf_meta, bf):
    return pl.pallas_call(
        _tail_body,
        out_shape=jax.ShapeDtypeStruct((1, N_OUT), jnp.float32),
    )(ri_row, bi, Wf_main, meta, Wf_meta, bf)


def kernel(x1, edge_index1, x2, edge_index2, meta, W1, b1, W2, b2, W3, b3,
           W4, b4, Wi, bi, Wf, bf):
    c1 = (b1 @ W2 @ W3 @ W4)[0]
    c2 = (b2 @ W3 @ W4)[0]
    c3 = (b3 @ W4)[0]
    c4 = b4[0]
    cvec = jnp.concatenate([jnp.full((LANES,), c, jnp.float32)
                            for c in (c1, c2, c3, c4)])

    v1, v2 = _project(x1, x2, W1, W2, W3, W4)
    vflat = jnp.concatenate([v1[:, 0], v2[:, 0]])
    src2 = jnp.concatenate([edge_index1[0], edge_index2[0]])
    dst2 = jnp.concatenate([edge_index1[1], edge_index2[1]])

    z2d = _gcn_sc()(vflat, src2, dst2, cvec)

    ri = _ri(Wi.T, z2d.reshape(2 * N_NODES, 1))
    out = _tail(ri.reshape(1, N_NODES), bi.reshape(1, N_NODES),
                Wf[:N_NODES], meta.reshape(1, N_META_IN), Wf[N_NODES:],
                bf.reshape(1, N_OUT))
    return out[0]
